# baseline (speedup 1.0000x reference)
import math

import jax
import jax.numpy as jnp
from jax import lax
from jax.experimental import pallas as pl
from jax.experimental.pallas import tpu as pltpu
from jax.experimental.pallas import tpu_sc as plsc

D_MODEL = 512
SCALE = math.sqrt(D_MODEL)

NUM_CORES = 2
NUM_SUBCORES = 16
NUM_LANES = 16
NW = NUM_CORES * NUM_SUBCORES

GROUP = 4


def _sc_embedding(x, table):
    b, t = x.shape
    assert b % (NW * GROUP) == 0 and (GROUP * t) % 8 == 0 and GROUP * t <= 128
    b_per_w = b // NW
    n_groups = b_per_w // GROUP
    gsz = GROUP * t
    mesh = plsc.VectorSubcoreMesh(core_axis_name="c", subcore_axis_name="s")

    def body(idx_hbm, table_hbm, out_hbm,
             idx_v, raw0, raw1, s0, s1, s2, gs0, gs1, ss0, ss1, ss2):
        wid = lax.axis_index("s") * NUM_CORES + lax.axis_index("c")
        pltpu.sync_copy(idx_hbm.at[wid], idx_v)
        base = wid * b_per_w
        last = n_groups - 1
        stg = [s0, s1, s2]
        ss = [ss0, ss1, ss2]

        def fire_gather(g, raw, sem):
            pltpu.make_async_copy(
                table_hbm.at[idx_v.at[pl.ds(g * gsz, gsz)]], raw, sem
            ).start()

        def drain_gather(raw, sem):
            pltpu.make_async_copy(
                table_hbm.at[idx_v.at[pl.ds(0, gsz)]], raw, sem
            ).wait()

        def scale_into(raw, h, k):
            def scale_row(r, _):
                for c in range(D_MODEL // NUM_LANES):
                    sl = pl.ds(c * NUM_LANES, NUM_LANES)
                    stg[k][r, sl] = raw[h * t + r, sl] * SCALE
                return 0

            lax.fori_loop(0, t, scale_row, 0)

        def fire_store(bb, k):
            pltpu.make_async_copy(stg[k], out_hbm.at[bb], ss[k]).start()

        def drain_store(k):
            pltpu.make_async_copy(stg[k], out_hbm.at[base], ss[k]).wait()

        def slot(g, raw, gsem, first):
            drain_gather(raw, gsem)
            for h in range(GROUP):
                k = h % len(stg)
                if (not first) or h >= len(stg):
                    drain_store(k)
                scale_into(raw, h, k)
                fire_store(base + g * GROUP + h, k)
            fire_gather(jnp.minimum(g + 2, last), raw, gsem)

        fire_gather(0, raw0, gs0)
        fire_gather(1, raw1, gs1)

        slot(0, raw0, gs0, True)
        slot(1, raw1, gs1, False)

        def ring(i, _):
            slot(2 * i, raw0, gs0, False)
            slot(2 * i + 1, raw1, gs1, False)
            return 0

        lax.fori_loop(1, n_groups // 2, ring, 0)
        for k in range(len(stg)):
            drain_store(k)
        drain_gather(raw0, gs0)
        drain_gather(raw1, gs1)

    run = pl.kernel(
        body,
        out_type=jax.ShapeDtypeStruct((b, t, D_MODEL), jnp.float32),
        mesh=mesh,
        scratch_types=[
            pltpu.VMEM((b_per_w * t,), jnp.int32),
            pltpu.VMEM((gsz, D_MODEL), jnp.float32),
            pltpu.VMEM((gsz, D_MODEL), jnp.float32),
            pltpu.VMEM((t, D_MODEL), jnp.float32),
            pltpu.VMEM((t, D_MODEL), jnp.float32),
            pltpu.VMEM((t, D_MODEL), jnp.float32),
            pltpu.SemaphoreType.DMA,
            pltpu.SemaphoreType.DMA,
            pltpu.SemaphoreType.DMA,
            pltpu.SemaphoreType.DMA,
            pltpu.SemaphoreType.DMA,
        ],
    )
    idx2 = x.astype(jnp.int32).reshape(NW, b_per_w * t)
    return run(idx2, table)


def kernel(x, table):
    return _sc_embedding(x, table)


# speedup vs baseline: 1.2229x; 1.2229x over previous
import math

import jax
import jax.numpy as jnp
from jax import lax
from jax.experimental import pallas as pl
from jax.experimental.pallas import tpu as pltpu
from jax.experimental.pallas import tpu_sc as plsc

D_MODEL = 512
SCALE = math.sqrt(D_MODEL)

NUM_CORES = 2
NUM_SUBCORES = 16
NUM_LANES = 16
NW = NUM_CORES * NUM_SUBCORES

GROUP = 2
N_RAW = 3
N_STG = 2
UNROLL = 6


def _sc_embedding(x, table):
    b, t = x.shape
    assert b % (NW * GROUP) == 0 and (GROUP * t) % 8 == 0 and GROUP * t <= 128
    b_per_w = b // NW
    n_groups = b_per_w // GROUP
    gsz = GROUP * t
    n_steady = (n_groups - UNROLL) // UNROLL
    n_tail = n_groups - UNROLL - n_steady * UNROLL
    mesh = plsc.VectorSubcoreMesh(core_axis_name="c", subcore_axis_name="s")

    def body(idx_hbm, table_hbm, out_hbm,
             idx_v, r0, r1, r2, s0, s1, gr0, gr1, gr2, sr0, sr1):
        wid = lax.axis_index("s") * NUM_CORES + lax.axis_index("c")
        pltpu.sync_copy(idx_hbm.at[wid], idx_v)
        base = wid * b_per_w
        last = n_groups - 1
        raw = [r0, r1, r2]
        gs = [gr0, gr1, gr2]
        stg = [s0, s1]
        ss = [sr0, sr1]

        def fire_gather(g, k):
            pltpu.make_async_copy(
                table_hbm.at[idx_v.at[pl.ds(g * gsz, gsz)]], raw[k], gs[k]
            ).start()

        def drain_gather(k):
            pltpu.make_async_copy(
                table_hbm.at[idx_v.at[pl.ds(0, gsz)]], raw[k], gs[k]
            ).wait()

        def scale_into(k, m):
            def scale_row(r, _):
                for j in range(GROUP):
                    for c in range(D_MODEL // NUM_LANES):
                        sl = pl.ds(c * NUM_LANES, NUM_LANES)
                        stg[m][j, r, sl] = raw[k][j * t + r, sl] * SCALE
                return 0

            lax.fori_loop(0, t, scale_row, 0)

        def fire_store(g, m):
            pltpu.make_async_copy(
                stg[m], out_hbm.at[pl.ds(base + g * GROUP, GROUP)], ss[m]
            ).start()

        def drain_store(m):
            pltpu.make_async_copy(
                stg[m], out_hbm.at[pl.ds(base, GROUP)], ss[m]
            ).wait()

        def slot(g, k, m, store_drain=True):
            drain_gather(k)
            if store_drain:
                drain_store(m)
            scale_into(k, m)
            fire_store(g, m)
            fire_gather(jnp.minimum(g + N_RAW, last), k)

        for k in range(N_RAW):
            fire_gather(k, k)

        for g in range(UNROLL):
            slot(g, g % N_RAW, g % N_STG, store_drain=(g >= N_STG))

        def ring(i, _):
            g0 = i * UNROLL
            for d in range(UNROLL):
                slot(g0 + d, d % N_RAW, d % N_STG)
            return 0

        lax.fori_loop(1, 1 + n_steady, ring, 0)

        g0 = (1 + n_steady) * UNROLL
        for d in range(n_tail):
            slot(g0 + d, (g0 + d) % N_RAW, (g0 + d) % N_STG)

        for m in range(N_STG):
            drain_store(m)
        for k in range(N_RAW):
            drain_gather(k)

    run = pl.kernel(
        body,
        out_type=jax.ShapeDtypeStruct((b, t, D_MODEL), jnp.float32),
        mesh=mesh,
        scratch_types=[
            pltpu.VMEM((b_per_w * t,), jnp.int32),
            pltpu.VMEM((gsz, D_MODEL), jnp.float32),
            pltpu.VMEM((gsz, D_MODEL), jnp.float32),
            pltpu.VMEM((gsz, D_MODEL), jnp.float32),
            pltpu.VMEM((GROUP, t, D_MODEL), jnp.float32),
            pltpu.VMEM((GROUP, t, D_MODEL), jnp.float32),
            pltpu.SemaphoreType.DMA,
            pltpu.SemaphoreType.DMA,
            pltpu.SemaphoreType.DMA,
            pltpu.SemaphoreType.DMA,
            pltpu.SemaphoreType.DMA,
        ],
    )
    idx2 = x.astype(jnp.int32).reshape(NW, b_per_w * t)
    return run(idx2, table)


def kernel(x, table):
    return _sc_embedding(x, table)


# speedup vs baseline: 1.2636x; 1.0333x over previous
import math

import jax
import jax.numpy as jnp
from jax import lax
from jax.experimental import pallas as pl
from jax.experimental.pallas import tpu as pltpu
from jax.experimental.pallas import tpu_sc as plsc

D_MODEL = 512
SCALE = math.sqrt(D_MODEL)

NUM_CORES = 2
NUM_SUBCORES = 16
NUM_LANES = 16
NW = NUM_CORES * NUM_SUBCORES

GROUP = 2


def _sc_embedding(x, table):
    b, t = x.shape
    assert b % (NW * GROUP) == 0 and (GROUP * t) % 8 == 0 and GROUP * t <= 128
    b_per_w = b // NW
    n_groups = b_per_w // GROUP
    gsz = GROUP * t
    mesh = plsc.VectorSubcoreMesh(core_axis_name="c", subcore_axis_name="s")

    def body(idx_hbm, table_hbm, out_hbm,
             idx_v, raw0, raw1, stg0, stg1, gs0, gs1, st0, st1):
        wid = lax.axis_index("s") * NUM_CORES + lax.axis_index("c")
        pltpu.sync_copy(idx_hbm.at[wid], idx_v)
        base = wid * b_per_w
        last = n_groups - 1

        def fire_gather(g, raw, sem):
            pltpu.make_async_copy(
                table_hbm.at[idx_v.at[pl.ds(g * gsz, gsz)]], raw, sem
            ).start()

        def drain_gather(raw, sem):
            pltpu.make_async_copy(
                table_hbm.at[idx_v.at[pl.ds(0, gsz)]], raw, sem
            ).wait()

        def scale_into(raw, stg):
            def scale_row(r, _):
                for j in range(GROUP):
                    for c in range(D_MODEL // NUM_LANES):
                        sl = pl.ds(c * NUM_LANES, NUM_LANES)
                        stg[j, r, sl] = raw[j * t + r, sl] * SCALE
                return 0

            lax.fori_loop(0, t, scale_row, 0)

        def fire_store(g, stg, sem):
            pltpu.make_async_copy(
                stg, out_hbm.at[pl.ds(base + g * GROUP, GROUP)], sem
            ).start()

        def drain_store(stg, sem):
            pltpu.make_async_copy(
                stg, out_hbm.at[pl.ds(base, GROUP)], sem
            ).wait()

        def slot(g, raw, stg, gsem, ssem, first):
            drain_gather(raw, gsem)
            if not first:
                drain_store(stg, ssem)
            scale_into(raw, stg)
            fire_store(g, stg, ssem)
            fire_gather(jnp.minimum(g + 2, last), raw, gsem)

        fire_gather(0, raw0, gs0)
        fire_gather(1, raw1, gs1)

        slot(0, raw0, stg0, gs0, st0, True)
        slot(1, raw1, stg1, gs1, st1, True)

        def ring(i, _):
            slot(2 * i, raw0, stg0, gs0, st0, False)
            slot(2 * i + 1, raw1, stg1, gs1, st1, False)
            return 0

        lax.fori_loop(1, n_groups // 2, ring, 0)
        drain_store(stg0, st0)
        drain_store(stg1, st1)
        drain_gather(raw0, gs0)
        drain_gather(raw1, gs1)

    run = pl.kernel(
        body,
        out_type=jax.ShapeDtypeStruct((b, t, D_MODEL), jnp.float32),
        mesh=mesh,
        scratch_types=[
            pltpu.VMEM((b_per_w * t,), jnp.int32),
            pltpu.VMEM((gsz, D_MODEL), jnp.float32),
            pltpu.VMEM((gsz, D_MODEL), jnp.float32),
            pltpu.VMEM((GROUP, t, D_MODEL), jnp.float32),
            pltpu.VMEM((GROUP, t, D_MODEL), jnp.float32),
            pltpu.SemaphoreType.DMA,
            pltpu.SemaphoreType.DMA,
            pltpu.SemaphoreType.DMA,
            pltpu.SemaphoreType.DMA,
        ],
    )
    idx2 = x.astype(jnp.int32).reshape(NW, b_per_w * t)
    return run(idx2, table)


def kernel(x, table):
    return _sc_embedding(x, table)


# speedup vs baseline: 1.6320x; 1.2916x over previous
import math

import jax
import jax.numpy as jnp
from jax import lax
from jax.experimental import pallas as pl
from jax.experimental.pallas import tpu as pltpu
from jax.experimental.pallas import tpu_sc as plsc

D_MODEL = 512
SCALE = math.sqrt(D_MODEL)

NUM_CORES = 2
NUM_SUBCORES = 16
NUM_LANES = 16
NW = NUM_CORES * NUM_SUBCORES

GROUP = 2


def _sc_embedding(x, table):
    b, t = x.shape
    assert b % (NW * GROUP) == 0 and (GROUP * t) % 8 == 0 and GROUP * t <= 128
    b_per_w = b // NW
    n_groups = b_per_w // GROUP
    gsz = GROUP * t
    mesh = plsc.VectorSubcoreMesh(core_axis_name="c", subcore_axis_name="s")

    def body(idx_hbm, table_hbm, out_hbm,
             idx_v, raw0, raw1, stg0, stg1, gs0, gs1, st0, st1):
        wid = lax.axis_index("s") * NUM_CORES + lax.axis_index("c")
        pltpu.sync_copy(idx_hbm.at[wid], idx_v)
        base = wid * b_per_w
        last = n_groups - 1

        def fire_gather(g, raw, sem):
            pltpu.make_async_copy(
                table_hbm.at[idx_v.at[pl.ds(g * gsz, gsz)]], raw, sem
            ).start()

        def drain_gather(raw, sem):
            pltpu.make_async_copy(
                table_hbm.at[idx_v.at[pl.ds(0, gsz)]], raw, sem
            ).wait()

        def scale_into(raw, stg):
            @plsc.parallel_loop(0, t)
            def scale_row(r):
                for j in range(GROUP):
                    for c in range(D_MODEL // NUM_LANES):
                        sl = pl.ds(c * NUM_LANES, NUM_LANES)
                        stg[j, r, sl] = raw[j * t + r, sl] * SCALE

        def fire_store(g, stg, sem):
            pltpu.make_async_copy(
                stg, out_hbm.at[pl.ds(base + g * GROUP, GROUP)], sem
            ).start()

        def drain_store(stg, sem):
            pltpu.make_async_copy(
                stg, out_hbm.at[pl.ds(base, GROUP)], sem
            ).wait()

        def slot(g, raw, stg, gsem, ssem, first):
            drain_gather(raw, gsem)
            if not first:
                drain_store(stg, ssem)
            scale_into(raw, stg)
            fire_store(g, stg, ssem)
            fire_gather(jnp.minimum(g + 2, last), raw, gsem)

        fire_gather(0, raw0, gs0)
        fire_gather(1, raw1, gs1)

        slot(0, raw0, stg0, gs0, st0, True)
        slot(1, raw1, stg1, gs1, st1, True)

        def ring(i, _):
            slot(2 * i, raw0, stg0, gs0, st0, False)
            slot(2 * i + 1, raw1, stg1, gs1, st1, False)
            return 0

        lax.fori_loop(1, n_groups // 2, ring, 0)
        drain_store(stg0, st0)
        drain_store(stg1, st1)
        drain_gather(raw0, gs0)
        drain_gather(raw1, gs1)

    run = pl.kernel(
        body,
        out_type=jax.ShapeDtypeStruct((b, t, D_MODEL), jnp.float32),
        mesh=mesh,
        scratch_types=[
            pltpu.VMEM((b_per_w * t,), jnp.int32),
            pltpu.VMEM((gsz, D_MODEL), jnp.float32),
            pltpu.VMEM((gsz, D_MODEL), jnp.float32),
            pltpu.VMEM((GROUP, t, D_MODEL), jnp.float32),
            pltpu.VMEM((GROUP, t, D_MODEL), jnp.float32),
            pltpu.SemaphoreType.DMA,
            pltpu.SemaphoreType.DMA,
            pltpu.SemaphoreType.DMA,
            pltpu.SemaphoreType.DMA,
        ],
    )
    idx2 = x.astype(jnp.int32).reshape(NW, b_per_w * t)
    return run(idx2, table)


def kernel(x, table):
    return _sc_embedding(x, table)
